# baseline (speedup 1.0000x reference)
import jax
import jax.numpy as jnp
from jax import lax
from jax.experimental import pallas as pl
from jax.experimental.pallas import tpu as pltpu


def _make_body(L, T, B, H, N, EP):
    G = 4 * H
    G2 = 2 * G
    H2 = 2 * H

    def body(tok_ref, emb_hbm,
             wih0_ref, b0_ref, whh0_ref,
             wih1_ref, b1_ref, whh1_ref,
             h0_ref, c0_ref, fcw_ref, fcb_ref,
             hn_ref, cn_ref, p_ref,
             x3_ref, gin0_ref, gin1_ref, act_ref, sems):
        groups = [(0, T - 1), (1, T - 2), tuple(range(2, T - 2))]

        def issue(group):
            for t in groups[group]:
                for b in range(B):
                    src = tok_ref[b * T + t]
                    pltpu.make_async_copy(emb_hbm.at[src],
                                          x3_ref.at[t * B + b],
                                          sems.at[group]).start()

        issue(0)
        issue(1)
        issue(2)

        f32 = jnp.float32

        def gate_scale(rows, cols):
            r = lax.broadcasted_iota(jnp.int32, (rows, cols), 0)
            return jnp.where((r % G >= 2 * H) & (r % G < 3 * H), 1.0, 0.5)

        sG = gate_scale(G, H)
        wT = []
        for w_ref in (whh0_ref, whh1_ref):
            w = w_ref[...]
            wT.append((jnp.swapaxes(w[:, :G], 0, 1) * sG,
                       jnp.swapaxes(w[:, G:], 0, 1) * sG))
        wih0T = jnp.swapaxes(wih0_ref[...], 0, 1) * gate_scale(G2, EP)
        wih1T = jnp.swapaxes(wih1_ref[...], 0, 1) * gate_scale(G2, H2)
        sB = gate_scale(G2, B)
        b0T = jnp.broadcast_to(
            jnp.swapaxes(b0_ref[...], 0, 1), (G2, B)) * sB
        b1T = jnp.broadcast_to(
            jnp.swapaxes(b1_ref[...], 0, 1), (G2, B)) * sB
        fcw0T = jnp.swapaxes(fcw_ref[:, 0:1], 0, 1)
        fcb0 = fcb_ref[:, 0:1]
        hs = [jnp.swapaxes(h0_ref[i], 0, 1) for i in range(2 * L)]
        cs = [jnp.swapaxes(c0_ref[i], 0, 1) for i in range(2 * L)]

        def step(g, c):
            th = jnp.tanh(g)
            i_g = th[0:H] * 0.5 + 0.5
            f_g = th[H:2 * H] * 0.5 + 0.5
            g_g = th[2 * H:3 * H]
            o_g = th[3 * H:] * 0.5 + 0.5
            c = f_g * c + i_g * g_g
            return o_g * jnp.tanh(c), c

        def wait_rows(group, nrows):
            pltpu.make_async_copy(x3_ref.at[pl.ds(0, nrows)],
                                  x3_ref.at[pl.ds(0, nrows)],
                                  sems.at[group]).wait()

        def gin0_block(t):
            xb = x3_ref[t * B:(t + 1) * B].reshape(B, EP)
            blkT = lax.dot_general(wih0T, xb, (((1,), (1,)), ((), ())),
                                   preferred_element_type=f32)
            gin0_ref[t * G2:(t + 1) * G2, :] = blkT + b0T

        def gin1_block(t):
            gin1_ref[t * G2:(t + 1) * G2, :] = (
                jnp.dot(wih1T, act_ref[t * H2:(t + 1) * H2, :],
                        preferred_element_type=f32) + b1T)

        wfT, wbT = wT[0]
        hf, cf = hs[0], cs[0]
        hb, cb = hs[1], cs[1]
        wait_rows(0, 2 * B)
        gin0_block(0)
        gin0_block(T - 1)
        for t in range(T):
            rt = T - 1 - t
            if t == 1:
                wait_rows(1, 2 * B)
                gin0_block(1)
                gin0_block(T - 2)
            elif 2 <= t <= T // 2 - 1:
                if t == 2:
                    wait_rows(2, (T - 4) * B)
                gin0_block(t)
                gin0_block(T - 1 - t)
            gf = (gin0_ref[t * G2:t * G2 + G, :]
                  + jnp.dot(wfT, hf, preferred_element_type=f32))
            gb = (gin0_ref[rt * G2 + G:(rt + 1) * G2, :]
                  + jnp.dot(wbT, hb, preferred_element_type=f32))
            hf, cf = step(gf, cf)
            hb, cb = step(gb, cb)
            act_ref[t * H2:t * H2 + H, :] = hf
            act_ref[rt * H2 + H:(rt + 1) * H2, :] = hb
            if t >= T // 2:
                gin1_block(rt)
                gin1_block(t)

        hn_ref[0] = jnp.swapaxes(hf, 0, 1)
        hn_ref[1] = jnp.swapaxes(hb, 0, 1)
        cn_ref[0] = jnp.swapaxes(cf, 0, 1)
        cn_ref[1] = jnp.swapaxes(cb, 0, 1)

        wfT, wbT = wT[1]
        hf, cf = hs[2], cs[2]
        hb, cb = hs[3], cs[3]
        last_bwd = None
        for t in range(T):
            rt = T - 1 - t
            gf = (gin1_ref[t * G2:t * G2 + G, :]
                  + jnp.dot(wfT, hf, preferred_element_type=f32))
            gb = (gin1_ref[rt * G2 + G:(rt + 1) * G2, :]
                  + jnp.dot(wbT, hb, preferred_element_type=f32))
            hf, cf = step(gf, cf)
            hb, cb = step(gb, cb)
            if t == 0:
                last_bwd = hb

        hn_ref[2] = jnp.swapaxes(hf, 0, 1)
        hn_ref[3] = jnp.swapaxes(hb, 0, 1)
        cn_ref[2] = jnp.swapaxes(cf, 0, 1)
        cn_ref[3] = jnp.swapaxes(cb, 0, 1)

        p_ref[...] = 0.5 * jnp.tanh(
            0.5 * (jnp.dot(fcw0T, last_bwd, preferred_element_type=f32)
                   + fcb0)) + 0.5

    return body


def kernel(tokens, h0, c0, embedding_pad, wih_0, whh_0, bias_0,
           wih_1, whh_1, bias_1, fc_w_pad, fc_b_pad):
    B, T = tokens.shape
    H = whh_0.shape[0]
    L = 2
    N = T * B
    vocab_p, EP = embedding_pad.shape

    tok = tokens.reshape(N).astype(jnp.int32)
    emb3 = embedding_pad.reshape(vocab_p, 1, EP)

    full = lambda shape: pl.BlockSpec(shape, lambda i, *_: (0,) * len(shape))
    grid_spec = pltpu.PrefetchScalarGridSpec(
        num_scalar_prefetch=1,
        grid=(1,),
        in_specs=[
            pl.BlockSpec(memory_space=pl.ANY),
            full(wih_0.shape), full(bias_0.shape), full(whh_0.shape),
            full(wih_1.shape), full(bias_1.shape), full(whh_1.shape),
            full(h0.shape), full(c0.shape),
            full(fc_w_pad.shape), full(fc_b_pad.shape),
        ],
        out_specs=(
            full((2 * L, B, H)),
            full((2 * L, B, H)),
            full((1, B)),
        ),
        scratch_shapes=[
            pltpu.VMEM((N, 1, EP), jnp.float32),
            pltpu.VMEM((T * 8 * H * 2, B), jnp.float32),
            pltpu.VMEM((T * 8 * H * 2, B), jnp.float32),
            pltpu.VMEM((T * 2 * H, B), jnp.float32),
            pltpu.SemaphoreType.DMA((3,)),
        ],
    )
    out_shape = (
        jax.ShapeDtypeStruct((2 * L, B, H), jnp.float32),
        jax.ShapeDtypeStruct((2 * L, B, H), jnp.float32),
        jax.ShapeDtypeStruct((1, B), jnp.float32),
    )
    hn, cn, p = pl.pallas_call(
        _make_body(L, T, B, H, N, EP),
        out_shape=out_shape,
        grid_spec=grid_spec,
        compiler_params=pltpu.CompilerParams(
            dimension_semantics=("arbitrary",),
            disable_bounds_checks=True),
    )(tok, emb3, wih_0, bias_0, whh_0, wih_1, bias_1, whh_1,
      h0, c0, fc_w_pad, fc_b_pad)

    return p.reshape(B), (hn, cn)


# speedup vs baseline: 1.0276x; 1.0276x over previous
import jax
import jax.numpy as jnp
from jax import lax
from jax.experimental import pallas as pl
from jax.experimental.pallas import tpu as pltpu


def _make_body(L, T, B, H, N, EP):
    G = 4 * H
    G2 = 2 * G
    H2 = 2 * H

    def body(tok_ref, emb_hbm,
             wih0_ref, b0_ref, whh0_ref,
             wih1_ref, b1_ref, whh1_ref,
             h0_ref, c0_ref, fcw_ref, fcb_ref,
             hn_ref, cn_ref, p_ref,
             x3_ref, gin0_ref, gin1_ref, act_ref, sems):
        groups = [(0, T - 1), (1, T - 2), tuple(range(2, T - 2))]

        def issue(group):
            for t in groups[group]:
                for b in range(B):
                    src = tok_ref[b * T + t]
                    pltpu.make_async_copy(emb_hbm.at[src],
                                          x3_ref.at[t * B + b],
                                          sems.at[group]).start()

        issue(0)
        issue(1)
        issue(2)

        f32 = jnp.float32

        def gate_scale(rows, cols):
            r = lax.broadcasted_iota(jnp.int32, (rows, cols), 0)
            return jnp.where((r % G >= 2 * H) & (r % G < 3 * H), 1.0, 0.5)

        sG = gate_scale(G, H)
        wT = []
        for w_ref in (whh0_ref, whh1_ref):
            w = w_ref[...]
            wT.append((jnp.swapaxes(w[:, :G], 0, 1) * sG,
                       jnp.swapaxes(w[:, G:], 0, 1) * sG))
        wih0T = jnp.swapaxes(wih0_ref[...], 0, 1) * gate_scale(G2, EP)
        wih1T = jnp.swapaxes(wih1_ref[...], 0, 1) * gate_scale(G2, H2)
        sB = gate_scale(G2, B)
        b0T = jnp.broadcast_to(
            jnp.swapaxes(b0_ref[...], 0, 1), (G2, B)) * sB
        b1T = jnp.broadcast_to(
            jnp.swapaxes(b1_ref[...], 0, 1), (G2, B)) * sB
        fcw0T = jnp.swapaxes(fcw_ref[:, 0:1], 0, 1)
        fcb0 = fcb_ref[:, 0:1]
        hs = [jnp.swapaxes(h0_ref[i], 0, 1) for i in range(2 * L)]
        cs = [jnp.swapaxes(c0_ref[i], 0, 1) for i in range(2 * L)]

        def step(g, c):
            th = jnp.tanh(g)
            i_g = th[0:H] * 0.5 + 0.5
            f_g = th[H:2 * H] * 0.5 + 0.5
            g_g = th[2 * H:3 * H]
            o_g = th[3 * H:] * 0.5 + 0.5
            c = f_g * c + i_g * g_g
            return o_g * jnp.tanh(c), c

        def wait_rows(group, nrows):
            pltpu.make_async_copy(x3_ref.at[pl.ds(0, nrows)],
                                  x3_ref.at[pl.ds(0, nrows)],
                                  sems.at[group]).wait()

        def gin0_block(t):
            xb = x3_ref[t * B:(t + 1) * B].reshape(B, EP)
            blkT = lax.dot_general(wih0T, xb, (((1,), (1,)), ((), ())),
                                   preferred_element_type=f32)
            gin0_ref[t * G2:(t + 1) * G2, :] = blkT + b0T

        def gin1_block(t):
            gin1_ref[t * G2:(t + 1) * G2, :] = (
                jnp.dot(wih1T, act_ref[t * H2:(t + 1) * H2, :],
                        preferred_element_type=f32) + b1T)

        wfT, wbT = wT[0]
        hf, cf = hs[0], cs[0]
        hb, cb = hs[1], cs[1]
        wait_rows(0, 2 * B)
        gin0_block(0)
        gin0_block(T - 1)
        for t in range(T):
            rt = T - 1 - t
            if t == 1:
                wait_rows(1, 2 * B)
                gin0_block(1)
                gin0_block(T - 2)
            elif 2 <= t <= T // 2 - 1:
                if t == 2:
                    wait_rows(2, (T - 4) * B)
                gin0_block(t)
                gin0_block(T - 1 - t)
            gf = (gin0_ref[t * G2:t * G2 + G, :]
                  + jnp.dot(wfT, hf, preferred_element_type=f32))
            gb = (gin0_ref[rt * G2 + G:(rt + 1) * G2, :]
                  + jnp.dot(wbT, hb, preferred_element_type=f32))
            hf, cf = step(gf, cf)
            hb, cb = step(gb, cb)
            act_ref[t * H2:t * H2 + H, :] = hf
            act_ref[rt * H2 + H:(rt + 1) * H2, :] = hb

        for t in range(T):
            gin1_block(t)

        hn_ref[0] = jnp.swapaxes(hf, 0, 1)
        hn_ref[1] = jnp.swapaxes(hb, 0, 1)
        cn_ref[0] = jnp.swapaxes(cf, 0, 1)
        cn_ref[1] = jnp.swapaxes(cb, 0, 1)

        wfT, wbT = wT[1]
        hf, cf = hs[2], cs[2]
        hb, cb = hs[3], cs[3]
        last_bwd = None
        for t in range(T):
            rt = T - 1 - t
            gf = (gin1_ref[t * G2:t * G2 + G, :]
                  + jnp.dot(wfT, hf, preferred_element_type=f32))
            gb = (gin1_ref[rt * G2 + G:(rt + 1) * G2, :]
                  + jnp.dot(wbT, hb, preferred_element_type=f32))
            hf, cf = step(gf, cf)
            hb, cb = step(gb, cb)
            if t == 0:
                last_bwd = hb

        hn_ref[2] = jnp.swapaxes(hf, 0, 1)
        hn_ref[3] = jnp.swapaxes(hb, 0, 1)
        cn_ref[2] = jnp.swapaxes(cf, 0, 1)
        cn_ref[3] = jnp.swapaxes(cb, 0, 1)

        p_ref[...] = 0.5 * jnp.tanh(
            0.5 * (jnp.dot(fcw0T, last_bwd, preferred_element_type=f32)
                   + fcb0)) + 0.5

    return body


def kernel(tokens, h0, c0, embedding_pad, wih_0, whh_0, bias_0,
           wih_1, whh_1, bias_1, fc_w_pad, fc_b_pad):
    B, T = tokens.shape
    H = whh_0.shape[0]
    L = 2
    N = T * B
    vocab_p, EP = embedding_pad.shape

    tok = tokens.reshape(N).astype(jnp.int32)
    emb3 = embedding_pad.reshape(vocab_p, 1, EP)

    full = lambda shape: pl.BlockSpec(shape, lambda i, *_: (0,) * len(shape))
    grid_spec = pltpu.PrefetchScalarGridSpec(
        num_scalar_prefetch=1,
        grid=(1,),
        in_specs=[
            pl.BlockSpec(memory_space=pl.ANY),
            full(wih_0.shape), full(bias_0.shape), full(whh_0.shape),
            full(wih_1.shape), full(bias_1.shape), full(whh_1.shape),
            full(h0.shape), full(c0.shape),
            full(fc_w_pad.shape), full(fc_b_pad.shape),
        ],
        out_specs=(
            full((2 * L, B, H)),
            full((2 * L, B, H)),
            full((1, B)),
        ),
        scratch_shapes=[
            pltpu.VMEM((N, 1, EP), jnp.float32),
            pltpu.VMEM((T * 8 * H * 2, B), jnp.float32),
            pltpu.VMEM((T * 8 * H * 2, B), jnp.float32),
            pltpu.VMEM((T * 2 * H, B), jnp.float32),
            pltpu.SemaphoreType.DMA((3,)),
        ],
    )
    out_shape = (
        jax.ShapeDtypeStruct((2 * L, B, H), jnp.float32),
        jax.ShapeDtypeStruct((2 * L, B, H), jnp.float32),
        jax.ShapeDtypeStruct((1, B), jnp.float32),
    )
    hn, cn, p = pl.pallas_call(
        _make_body(L, T, B, H, N, EP),
        out_shape=out_shape,
        grid_spec=grid_spec,
        compiler_params=pltpu.CompilerParams(
            dimension_semantics=("arbitrary",),
            disable_bounds_checks=True),
    )(tok, emb3, wih_0, bias_0, whh_0, wih_1, bias_1, whh_1,
      h0, c0, fc_w_pad, fc_b_pad)

    return p.reshape(B), (hn, cn)
